# Initial kernel scaffold; baseline (speedup 1.0000x reference)
import jax
import jax.numpy as jnp
from jax.experimental import pallas as pl


def kernel(atom_features, edge_index, batch, W1, b1, gamma, beta, W2, b2):
    raise NotImplementedError("write your pallas kernel here")


# speedup vs baseline: 32.0415x; 32.0415x over previous
import jax
import jax.numpy as jnp
from jax import lax
from jax.experimental import pallas as pl
from jax.experimental.pallas import tpu as pltpu
from jax.experimental.pallas import tpu_sc as plsc

N = 10000
E = 320000
NG = 64
NSUB = 16
NW = 32
EPT = E // NW
CH = 50
GC = 50
NGRP = EPT // (CH * GC)
NBUF = 5
AHEAD = 4
CHD = 100
NCHD = EPT // CHD
SLAB = 640
LAST = N - 15 * SLAB
DEGP = 10240
RB = 1000

_mesh = plsc.VectorSubcoreMesh(
    core_axis_name="c", subcore_axis_name="s", num_cores=2, num_subcores=NSUB)


def _deg_body(dst3, zeros1, ones1, out, dst_idx, ones_v, dsem, deg_sh):
    c = lax.axis_index("c")
    s = lax.axis_index("s")
    w = c * NSUB + s
    off = pl.multiple_of(s * 640, 8)
    pltpu.sync_copy(zeros1, deg_sh.at[pl.ds(off, 640)])
    pltpu.sync_copy(dst3.at[w], dst_idx)
    pltpu.sync_copy(ones1, ones_v)
    plsc.subcore_barrier()

    def chunk(j, carry):
        pltpu.async_copy(ones_v, deg_sh.at[dst_idx.at[j]], dsem, add=True)

        @pl.when(j >= 8)
        def _():
            pltpu.make_async_copy(
                ones_v, deg_sh.at[dst_idx.at[j - 8]], dsem).wait()

        return carry

    lax.fori_loop(0, NCHD, chunk, 0)

    def drain(j, carry):
        pltpu.make_async_copy(
            ones_v, deg_sh.at[dst_idx.at[NCHD - 8 + j]], dsem).wait()
        return carry

    lax.fori_loop(0, 8, drain, 0)
    plsc.subcore_barrier()
    pltpu.sync_copy(deg_sh.at[pl.ds(off, 640)], out.at[c, pl.ds(off, 640)])


_deg = pl.kernel(
    _deg_body,
    out_type=jax.ShapeDtypeStruct((2, DEGP), jnp.float32),
    mesh=_mesh,
    scratch_types=[
        pltpu.VMEM((NCHD, CHD), jnp.int32),
        pltpu.VMEM((CHD,), jnp.float32),
        pltpu.SemaphoreType.DMA,
        pltpu.VMEM_SHARED((DEGP,), jnp.float32),
    ],
)


def _seg_body(g_hbm, src2, dst2, zrows, out,
              src_idx, dst_idx, rows, gsem, ssem, acc_sh):
    c = lax.axis_index("c")
    s = lax.axis_index("s")
    w = c * NSUB + s
    row0 = pl.multiple_of(s * SLAB, 8)

    @pl.when(s < NSUB - 1)
    def _():
        pltpu.sync_copy(zrows, acc_sh.at[pl.ds(row0, SLAB)])

    @pl.when(s == NSUB - 1)
    def _():
        pltpu.sync_copy(zrows.at[pl.ds(0, LAST)],
                        acc_sh.at[pl.ds(15 * SLAB, LAST)])

    plsc.subcore_barrier()

    for g in range(NGRP):
        pltpu.sync_copy(src2.at[w, g], src_idx)
        pltpu.sync_copy(dst2.at[w, g], dst_idx)
        for b in range(AHEAD):
            pltpu.async_copy(g_hbm.at[src_idx.at[b]], rows.at[b], gsem.at[b])

        def step(i, carry):
            for b in range(NBUF):
                j = i * NBUF + b
                pltpu.make_async_copy(
                    g_hbm.at[src_idx.at[j]], rows.at[b], gsem.at[b]).wait()
                pltpu.async_copy(
                    rows.at[b], acc_sh.at[dst_idx.at[j]], ssem.at[b],
                    add=True)
                jn = j + AHEAD
                bn = (b + AHEAD) % NBUF

                @pl.when(jn < GC)
                def _():
                    @pl.when(jn >= NBUF)
                    def _():
                        pltpu.make_async_copy(
                            rows.at[bn], acc_sh.at[dst_idx.at[jn - NBUF]],
                            ssem.at[bn]).wait()

                    pltpu.async_copy(
                        g_hbm.at[src_idx.at[jn]], rows.at[bn], gsem.at[bn])

            return carry

        lax.fori_loop(0, GC // NBUF, step, 0)
        for b in range(NBUF):
            j = GC - NBUF + b
            pltpu.make_async_copy(
                rows.at[b], acc_sh.at[dst_idx.at[j]], ssem.at[b]).wait()

    plsc.subcore_barrier()

    @pl.when(s < NSUB - 1)
    def _():
        pltpu.sync_copy(acc_sh.at[pl.ds(row0, SLAB)],
                        out.at[c, pl.ds(row0, SLAB)])

    @pl.when(s == NSUB - 1)
    def _():
        pltpu.sync_copy(acc_sh.at[pl.ds(15 * SLAB, LAST)],
                        out.at[c, pl.ds(15 * SLAB, LAST)])


_seg = pl.kernel(
    _seg_body,
    out_type=jax.ShapeDtypeStruct((2, N, 128), jnp.float32),
    mesh=_mesh,
    scratch_types=[
        pltpu.VMEM((GC, CH), jnp.int32),
        pltpu.VMEM((GC, CH), jnp.int32),
        pltpu.VMEM((NBUF, CH, 128), jnp.float32),
        pltpu.SemaphoreType.DMA((NBUF,)),
        pltpu.SemaphoreType.DMA((NBUF,)),
        pltpu.VMEM_SHARED((N, 128), jnp.float32),
    ],
)


def _dinv(degt_ref):
    d = degt_ref[:, 0:1] + degt_ref[:, 1:2] + 1.0
    return lax.rsqrt(d)


def _tc1_body(x_ref, w1_ref, degt_ref, g1_ref):
    h = jnp.dot(x_ref[...], w1_ref[...], preferred_element_type=jnp.float32)
    g1_ref[...] = h * _dinv(degt_ref)


def _tc2_body(s1_ref, g1_ref, degt_ref, b1_ref, gam_ref, bet_ref, w2_ref,
              g2_ref, z_scr, st_scr):
    t = pl.program_id(0)
    nb = N // RB
    p = t // nb
    i = lax.rem(t, nb)

    @pl.when(p == 0)
    def _():
        z = (_dinv(degt_ref) * (s1_ref[0] + s1_ref[1] + g1_ref[...])
             + b1_ref[...])
        z_scr[pl.ds(i * RB, RB), :] = z
        cs = jnp.sum(z, axis=0, keepdims=True)
        cq = jnp.sum(z * z, axis=0, keepdims=True)
        upd = jnp.concatenate(
            [cs, cq, jnp.zeros((6, 128), jnp.float32)], axis=0)

        @pl.when(i == 0)
        def _():
            st_scr[...] = upd

        @pl.when(i != 0)
        def _():
            st_scr[...] = st_scr[...] + upd

    @pl.when(p == 1)
    def _():
        mean = st_scr[0:1, :] * (1.0 / N)
        var = st_scr[1:2, :] * (1.0 / N) - mean * mean
        inv = lax.rsqrt(var + 1e-5)
        z = z_scr[pl.ds(i * RB, RB), :]
        zn = (z - mean) * inv * gam_ref[...] + bet_ref[...]
        r = jnp.maximum(zn, 0.0)
        h = jnp.dot(r, w2_ref[...], preferred_element_type=jnp.float32)
        g2_ref[...] = h * _dinv(degt_ref)


def _tc3_body(s2_ref, g2_ref, degt_ref, b2_ref, bat_ref, out_ref):
    i = pl.program_id(0)

    @pl.when(i == 0)
    def _():
        out_ref[...] = jnp.full((NG, 128), -jnp.inf, jnp.float32)

    h2 = _dinv(degt_ref) * (s2_ref[0] + s2_ref[1] + g2_ref[...]) + b2_ref[...]
    b = bat_ref[...]
    lo = jnp.min(b)
    hi = jnp.max(b)

    def body(gid, carry):
        m = b == gid
        row = jnp.max(jnp.where(m, h2, -jnp.inf), axis=0, keepdims=True)
        out_ref[pl.ds(gid, 1), :] = jnp.maximum(out_ref[pl.ds(gid, 1), :], row)
        return carry

    lax.fori_loop(lo, hi + 1, body, 0)


def _row_spec():
    return pl.BlockSpec((RB, 128), lambda i: (i, 0))


def _tc1(x, W1, degt):
    return pl.pallas_call(
        _tc1_body,
        grid=(N // RB,),
        in_specs=[
            _row_spec(),
            pl.BlockSpec((128, 128), lambda i: (0, 0)),
            pl.BlockSpec((RB, 2), lambda i: (i, 0)),
        ],
        out_specs=_row_spec(),
        out_shape=jax.ShapeDtypeStruct((N, 128), jnp.float32),
    )(x, W1, degt)


def _tc2(s1, g1, degt, b1r, gammar, betar, W2p):
    nb = N // RB

    def _row(t):
        return (lax.rem(t, nb), 0)

    return pl.pallas_call(
        _tc2_body,
        grid=(2 * nb,),
        in_specs=[
            pl.BlockSpec((2, RB, 128), lambda t: (0, lax.rem(t, nb), 0)),
            pl.BlockSpec((RB, 128), _row),
            pl.BlockSpec((RB, 2), _row),
            pl.BlockSpec((1, 128), lambda t: (0, 0)),
            pl.BlockSpec((1, 128), lambda t: (0, 0)),
            pl.BlockSpec((1, 128), lambda t: (0, 0)),
            pl.BlockSpec((128, 128), lambda t: (0, 0)),
        ],
        out_specs=pl.BlockSpec((RB, 128),
                               lambda t: (jnp.maximum(t - nb, 0), 0)),
        out_shape=jax.ShapeDtypeStruct((N, 128), jnp.float32),
        scratch_shapes=[
            pltpu.VMEM((N, 128), jnp.float32),
            pltpu.VMEM((8, 128), jnp.float32),
        ],
    )(s1, g1, degt, b1r, gammar, betar, W2p)


def _tc3(s2, g2, degt, b2r, bat):
    return pl.pallas_call(
        _tc3_body,
        grid=(N // RB,),
        in_specs=[
            pl.BlockSpec((2, RB, 128), lambda i: (0, i, 0)),
            _row_spec(),
            pl.BlockSpec((RB, 2), lambda i: (i, 0)),
            pl.BlockSpec((1, 128), lambda i: (0, 0)),
            pl.BlockSpec((RB, 1), lambda i: (i, 0)),
        ],
        out_specs=pl.BlockSpec((NG, 128), lambda i: (0, 0)),
        out_shape=jax.ShapeDtypeStruct((NG, 128), jnp.float32),
    )(s2, g2, degt, b2r, bat)


def kernel(atom_features, edge_index, batch, W1, b1, gamma, beta, W2, b2):
    src_i = edge_index[0].astype(jnp.int32)
    dst_i = edge_index[1].astype(jnp.int32)
    src4 = src_i.reshape(NW, NGRP, GC, CH)
    dst4 = dst_i.reshape(NW, NGRP, GC, CH)
    dst3d = dst_i.reshape(NW, NCHD, CHD)
    bat = batch.astype(jnp.int32).reshape(N, 1)

    zrows = jnp.zeros((SLAB, 128), jnp.float32)
    zeros1 = jnp.zeros((640,), jnp.float32)
    ones1 = jnp.ones((CHD,), jnp.float32)

    b1r = b1.reshape(1, 128)
    gammar = gamma.reshape(1, 128)
    betar = beta.reshape(1, 128)
    W2p = jnp.zeros((128, 128), jnp.float32).at[:, :100].set(W2)
    b2r = jnp.zeros((1, 128), jnp.float32).at[0, :100].set(b2)

    degp = _deg(dst3d, zeros1, ones1)
    degt = degp[:, :N].T

    g1 = _tc1(atom_features, W1, degt)
    s1 = _seg(g1, src4, dst4, zrows)
    g2 = _tc2(s1, g1, degt, b1r, gammar, betar, W2p)
    s2 = _seg(g2, src4, dst4, zrows)
    out = _tc3(s2, g2, degt, b2r, bat)
    return out[:, :100]
